# Initial kernel scaffold; baseline (speedup 1.0000x reference)
import jax
import jax.numpy as jnp
from jax.experimental import pallas as pl


def kernel(x, router_w, gate_w, up_w, down_w):
    raise NotImplementedError("write your pallas kernel here")


# speedup vs baseline: 1.6309x; 1.6309x over previous
import jax
import jax.numpy as jnp
from jax.experimental import pallas as pl
from jax.experimental.pallas import tpu as pltpu

_TI = 1408


def _moe_body(x_ref, rw_ref, g_ref, u_ref, d_ref, out_ref, scale_ref):
    e = pl.program_id(0)
    i = pl.program_id(1)
    first = jnp.logical_and(e == 0, i == 0)

    @pl.when(first)
    def _routing():
        x = x_ref[...]
        logits = jax.lax.dot_general(
            x, rw_ref[...], (((1,), (1,)), ((), ())),
            preferred_element_type=jnp.float32)
        t_, e_ = logits.shape
        lanes = jax.lax.broadcasted_iota(jnp.int32, (t_, e_), 1)
        big = jnp.int32(2 ** 30)
        m1 = jnp.max(logits, axis=1, keepdims=True)
        idx1 = jnp.min(jnp.where(logits == m1, lanes, big), axis=1, keepdims=True)
        sel1 = lanes == idx1
        rem = jnp.where(sel1, -jnp.inf, logits)
        m2 = jnp.max(rem, axis=1, keepdims=True)
        idx2 = jnp.min(jnp.where(rem == m2, lanes, big), axis=1, keepdims=True)
        sel2 = lanes == idx2
        scale_ref[...] = jnp.where(
            jnp.logical_or(sel1, sel2), 0.25, 0.0).astype(jnp.float32)

    x = x_ref[...]
    g = jax.lax.dot_general(x, g_ref[0], (((1,), (1,)), ((), ())),
                            preferred_element_type=jnp.float32)
    u = jax.lax.dot_general(x, u_ref[0], (((1,), (1,)), ((), ())),
                            preferred_element_type=jnp.float32)
    h = g * jax.lax.logistic(g) * u
    lanes = jax.lax.broadcasted_iota(jnp.int32, scale_ref.shape, 1)
    col = jnp.sum(jnp.where(lanes == e, scale_ref[...], 0.0),
                  axis=1, keepdims=True)
    h = h * col
    contrib = jax.lax.dot_general(h, d_ref[0], (((1,), (1,)), ((), ())),
                                  preferred_element_type=jnp.float32)

    @pl.when(first)
    def _init():
        out_ref[...] = contrib

    @pl.when(jnp.logical_not(first))
    def _acc():
        out_ref[...] += contrib


def kernel(x, router_w, gate_w, up_w, down_w):
    t, h = x.shape
    e, i_dim, _ = gate_w.shape
    ni = i_dim // _TI
    return pl.pallas_call(
        _moe_body,
        grid=(e, ni),
        in_specs=[
            pl.BlockSpec((t, h), lambda e_, i_: (0, 0)),
            pl.BlockSpec((e, h), lambda e_, i_: (0, 0)),
            pl.BlockSpec((1, _TI, h), lambda e_, i_: (e_, i_, 0)),
            pl.BlockSpec((1, _TI, h), lambda e_, i_: (e_, i_, 0)),
            pl.BlockSpec((1, h, _TI), lambda e_, i_: (e_, 0, i_)),
        ],
        out_specs=pl.BlockSpec((t, h), lambda e_, i_: (0, 0)),
        out_shape=jax.ShapeDtypeStruct((t, h), x.dtype),
        scratch_shapes=[pltpu.VMEM((t, e), jnp.float32)],
    )(x, router_w, gate_w, up_w, down_w)
